# Initial kernel scaffold; baseline (speedup 1.0000x reference)
import functools

import jax
import jax.numpy as jnp
from jax import lax
from jax.experimental import pallas as pl
from jax.experimental.pallas import tpu as pltpu
from jax.experimental.pallas import tpu_sc as plsc

N = 10000
E = 320000
D = 128
L = 3

NC = 2
NS = 16
NW = NC * NS
EPW = E // NW
CH = 80
NCH = EPW // CH
RPT = N // NS
ZR = 25

R = 1250


def _seg_body(src_hbm, dst_hbm, m_hbm, out_hbm, src_v, dst_v, rows_v, zero_v,
              agg_sh, sem):
    c = lax.axis_index("c")
    s = lax.axis_index("s")
    wid = c * NS + s

    zf = jnp.zeros((16,), jnp.float32)
    for i in range(ZR):
        for j in range(D // 16):
            zero_v[i, pl.ds(j * 16, 16)] = zf

    def zbody(r, carry):
        pltpu.sync_copy(zero_v, agg_sh.at[pl.ds(s * RPT + r * ZR, ZR)])
        return carry

    lax.fori_loop(0, RPT // ZR, zbody, 0)
    plsc.subcore_barrier()

    pltpu.sync_copy(src_hbm.at[wid], src_v)
    pltpu.sync_copy(dst_hbm.at[wid], dst_v)

    def body(jc, carry):
        pltpu.async_copy(m_hbm.at[src_v.at[jc]], rows_v, sem).wait()
        pltpu.sync_copy(rows_v, agg_sh.at[dst_v.at[jc]], add=True)
        return carry

    lax.fori_loop(0, NCH, body, 0)
    plsc.subcore_barrier()

    pltpu.sync_copy(agg_sh.at[pl.ds(s * RPT, RPT)],
                    out_hbm.at[c].at[pl.ds(s * RPT, RPT)])


_seg_sum = pl.kernel(
    _seg_body,
    out_type=jax.ShapeDtypeStruct((NC, N, D), jnp.float32),
    mesh=plsc.VectorSubcoreMesh(core_axis_name="c", subcore_axis_name="s"),
    scratch_types=[
        pltpu.VMEM((NCH, CH), jnp.int32),
        pltpu.VMEM((NCH, CH), jnp.int32),
        pltpu.VMEM((CH, D), jnp.float32),
        pltpu.VMEM((ZR, D), jnp.float32),
        pltpu.VMEM_SHARED((N, D), jnp.float32),
        pltpu.SemaphoreType.DMA,
    ],
)


def _prologue_body(x_ref, w0_ref, b0_ref, cw_ref, h_ref, m_ref):
    h = jnp.dot(x_ref[...], w0_ref[...], preferred_element_type=jnp.float32)
    h = jnp.maximum(h + b0_ref[...], 0.0)
    h_ref[...] = h
    m_ref[...] = jnp.dot(h, cw_ref[...], preferred_element_type=jnp.float32)


_prologue = pl.pallas_call(
    _prologue_body,
    grid=(N // R,),
    in_specs=[
        pl.BlockSpec((R, D), lambda i: (i, 0)),
        pl.BlockSpec((D, D), lambda i: (0, 0)),
        pl.BlockSpec((1, D), lambda i: (0, 0)),
        pl.BlockSpec((D, D), lambda i: (0, 0)),
    ],
    out_specs=[
        pl.BlockSpec((R, D), lambda i: (i, 0)),
        pl.BlockSpec((R, D), lambda i: (i, 0)),
    ],
    out_shape=[
        jax.ShapeDtypeStruct((N, D), jnp.float32),
        jax.ShapeDtypeStruct((N, D), jnp.float32),
    ],
)


def _step_body(p0_ref, p1_ref, h_ref, wih_ref, whh_ref, bih_ref, bhh_ref,
               wn_ref, bn_ref, ho_ref, mo_ref):
    agg = p0_ref[0] + p1_ref[0]
    h = h_ref[...]
    gi = jnp.dot(agg, wih_ref[...], preferred_element_type=jnp.float32)
    gi = gi + bih_ref[...]
    gh = jnp.dot(h, whh_ref[...], preferred_element_type=jnp.float32)
    gh = gh + bhh_ref[...]
    r = jax.nn.sigmoid(gi[:, :D] + gh[:, :D])
    z = jax.nn.sigmoid(gi[:, D:2 * D] + gh[:, D:2 * D])
    n = jnp.tanh(gi[:, 2 * D:] + r * gh[:, 2 * D:])
    hn = (1.0 - z) * n + z * h
    ho_ref[...] = hn
    mo_ref[...] = jnp.dot(hn, wn_ref[...], preferred_element_type=jnp.float32) + bn_ref[...]


_step = pl.pallas_call(
    _step_body,
    grid=(N // R,),
    in_specs=[
        pl.BlockSpec((1, R, D), lambda i: (0, i, 0)),
        pl.BlockSpec((1, R, D), lambda i: (1, i, 0)),
        pl.BlockSpec((R, D), lambda i: (i, 0)),
        pl.BlockSpec((D, 3 * D), lambda i: (0, 0)),
        pl.BlockSpec((D, 3 * D), lambda i: (0, 0)),
        pl.BlockSpec((1, 3 * D), lambda i: (0, 0)),
        pl.BlockSpec((1, 3 * D), lambda i: (0, 0)),
        pl.BlockSpec((D, D), lambda i: (0, 0)),
        pl.BlockSpec((1, D), lambda i: (0, 0)),
    ],
    out_specs=[
        pl.BlockSpec((R, D), lambda i: (i, 0)),
        pl.BlockSpec((R, D), lambda i: (i, 0)),
    ],
    out_shape=[
        jax.ShapeDtypeStruct((N, D), jnp.float32),
        jax.ShapeDtypeStruct((N, D), jnp.float32),
    ],
)


def kernel(x, edge_index, lin0_W, lin0_b, conv_weight, gru_Wih, gru_Whh,
           gru_bih, gru_bhh, lin1_W, lin1_b):
    src = edge_index[0].astype(jnp.int32).reshape(NW, NCH, CH)
    dst = edge_index[1].astype(jnp.int32).reshape(NW, NCH, CH)

    wih_t = jnp.transpose(gru_Wih, (0, 2, 1))
    whh_t = jnp.transpose(gru_Whh, (0, 2, 1))
    bih = gru_bih.reshape(L, 1, 3 * D)
    bhh = gru_bhh.reshape(L, 1, 3 * D)
    b0 = lin0_b.reshape(1, D)
    b1 = lin1_b.reshape(1, D)
    zb = jnp.zeros((1, D), jnp.float32)

    h, m = _prologue(x, lin0_W, b0, conv_weight[0, 0])
    for t in range(L * L):
        l = t // L
        parts = _seg_sum(src, dst, m)
        if t < L * L - 1:
            ln, kn = (t + 1) // L, (t + 1) % L
            wn, bn = conv_weight[ln, kn], zb
        else:
            wn, bn = lin1_W, b1
        h, m = _step(parts, parts, h, wih_t[l], whh_t[l], bih[l], bhh[l],
                     wn, bn)
    return m


# speedup vs baseline: 6.4817x; 6.4817x over previous
import functools

import jax
import jax.numpy as jnp
from jax import lax
from jax.experimental import pallas as pl
from jax.experimental.pallas import tpu as pltpu
from jax.experimental.pallas import tpu_sc as plsc

N = 10000
E = 320000
D = 128
L = 3

NC = 2
NS = 16
NW = NC * NS
EPW = E // NW
CH = 80
NCH = EPW // CH
NP = 10240
RPT = NP // NS
ZR = 32

R = 2000


def _seg_body(src_hbm, dst_hbm, m_hbm, out_hbm, src_v, dst_v, rows_v, zero_v,
              agg_sh, sem):
    c = lax.axis_index("c")
    s = lax.axis_index("s")
    wid = c * NS + s

    zf = jnp.zeros((16,), jnp.float32)
    for i in range(ZR):
        for j in range(D // 16):
            zero_v[i, pl.ds(j * 16, 16)] = zf

    def zbody(r, carry):
        pltpu.sync_copy(zero_v, agg_sh.at[pl.ds(s * RPT + r * ZR, ZR)])
        return carry

    lax.fori_loop(0, RPT // ZR, zbody, 0)
    plsc.subcore_barrier()

    pltpu.sync_copy(src_hbm.at[wid], src_v)
    pltpu.sync_copy(dst_hbm.at[wid], dst_v)

    def body(jc, carry):
        pltpu.async_copy(m_hbm.at[src_v.at[jc]], rows_v, sem).wait()
        pltpu.sync_copy(rows_v, agg_sh.at[dst_v.at[jc]], add=True)
        return carry

    lax.fori_loop(0, NCH, body, 0)
    plsc.subcore_barrier()

    pltpu.sync_copy(agg_sh.at[pl.ds(s * RPT, RPT)],
                    out_hbm.at[c].at[pl.ds(s * RPT, RPT)])


_seg_sum = pl.kernel(
    _seg_body,
    out_type=jax.ShapeDtypeStruct((NC, NP, D), jnp.float32),
    mesh=plsc.VectorSubcoreMesh(core_axis_name="c", subcore_axis_name="s"),
    scratch_types=[
        pltpu.VMEM((NCH, CH), jnp.int32),
        pltpu.VMEM((NCH, CH), jnp.int32),
        pltpu.VMEM((CH, D), jnp.float32),
        pltpu.VMEM((ZR, D), jnp.float32),
        pltpu.VMEM_SHARED((NP, D), jnp.float32),
        pltpu.SemaphoreType.DMA,
    ],
)


def _prologue_body(x_ref, w0_ref, b0_ref, cw_ref, h_ref, m_ref):
    h = jnp.dot(x_ref[...], w0_ref[...], preferred_element_type=jnp.float32)
    h = jnp.maximum(h + b0_ref[...], 0.0)
    h_ref[...] = h
    m_ref[...] = jnp.dot(h, cw_ref[...], preferred_element_type=jnp.float32)


_prologue = pl.pallas_call(
    _prologue_body,
    grid=(N // R,),
    in_specs=[
        pl.BlockSpec((R, D), lambda i: (i, 0)),
        pl.BlockSpec((D, D), lambda i: (0, 0)),
        pl.BlockSpec((1, D), lambda i: (0, 0)),
        pl.BlockSpec((D, D), lambda i: (0, 0)),
    ],
    out_specs=[
        pl.BlockSpec((R, D), lambda i: (i, 0)),
        pl.BlockSpec((R, D), lambda i: (i, 0)),
    ],
    out_shape=[
        jax.ShapeDtypeStruct((N, D), jnp.float32),
        jax.ShapeDtypeStruct((N, D), jnp.float32),
    ],
)


def _step_body(p0_ref, p1_ref, h_ref, wih_ref, whh_ref, bih_ref, bhh_ref,
               wn_ref, bn_ref, ho_ref, mo_ref):
    agg = p0_ref[0] + p1_ref[0]
    h = h_ref[...]
    gi = jnp.dot(agg, wih_ref[...], preferred_element_type=jnp.float32)
    gi = gi + bih_ref[...]
    gh = jnp.dot(h, whh_ref[...], preferred_element_type=jnp.float32)
    gh = gh + bhh_ref[...]
    r = jax.nn.sigmoid(gi[:, :D] + gh[:, :D])
    z = jax.nn.sigmoid(gi[:, D:2 * D] + gh[:, D:2 * D])
    n = jnp.tanh(gi[:, 2 * D:] + r * gh[:, 2 * D:])
    hn = (1.0 - z) * n + z * h
    ho_ref[...] = hn
    mo_ref[...] = jnp.dot(hn, wn_ref[...], preferred_element_type=jnp.float32) + bn_ref[...]


_step = pl.pallas_call(
    _step_body,
    grid=(N // R,),
    in_specs=[
        pl.BlockSpec((1, R, D), lambda i: (0, i, 0)),
        pl.BlockSpec((1, R, D), lambda i: (1, i, 0)),
        pl.BlockSpec((R, D), lambda i: (i, 0)),
        pl.BlockSpec((D, 3 * D), lambda i: (0, 0)),
        pl.BlockSpec((D, 3 * D), lambda i: (0, 0)),
        pl.BlockSpec((1, 3 * D), lambda i: (0, 0)),
        pl.BlockSpec((1, 3 * D), lambda i: (0, 0)),
        pl.BlockSpec((D, D), lambda i: (0, 0)),
        pl.BlockSpec((1, D), lambda i: (0, 0)),
    ],
    out_specs=[
        pl.BlockSpec((R, D), lambda i: (i, 0)),
        pl.BlockSpec((R, D), lambda i: (i, 0)),
    ],
    out_shape=[
        jax.ShapeDtypeStruct((N, D), jnp.float32),
        jax.ShapeDtypeStruct((N, D), jnp.float32),
    ],
)


def kernel(x, edge_index, lin0_W, lin0_b, conv_weight, gru_Wih, gru_Whh,
           gru_bih, gru_bhh, lin1_W, lin1_b):
    src = edge_index[0].astype(jnp.int32).reshape(NW, NCH, CH)
    dst = edge_index[1].astype(jnp.int32).reshape(NW, NCH, CH)

    wih_t = jnp.transpose(gru_Wih, (0, 2, 1))
    whh_t = jnp.transpose(gru_Whh, (0, 2, 1))
    bih = gru_bih.reshape(L, 1, 3 * D)
    bhh = gru_bhh.reshape(L, 1, 3 * D)
    b0 = lin0_b.reshape(1, D)
    b1 = lin1_b.reshape(1, D)
    zb = jnp.zeros((1, D), jnp.float32)

    h, m = _prologue(x, lin0_W, b0, conv_weight[0, 0])
    for t in range(L * L):
        l = t // L
        parts = _seg_sum(src, dst, m)
        if t < L * L - 1:
            ln, kn = (t + 1) // L, (t + 1) % L
            wn, bn = conv_weight[ln, kn], zb
        else:
            wn, bn = lin1_W, b1
        h, m = _step(parts, parts, h, wih_t[l], whh_t[l], bih[l], bhh[l],
                     wn, bn)
    return m
